# baseline (speedup 1.0000x reference)
import functools

import jax
import jax.numpy as jnp
from jax import lax
from jax.experimental import pallas as pl
from jax.experimental.pallas import tpu as pltpu
from jax.experimental.pallas import tpu_sc as plsc

T, N, E, D, H, O = 8, 10000, 160000, 256, 256, 128
NP = 10240
NBINS = 32
NB = 320
CAP = 5632
EP = 163840
CH = 128
CG = 32
K = 4
NCB = CAP // CG
OCT = 16
P2C = EP // 32 // CH
OPC = EP // 16 // CH
RPT = NP // 16
HD = D // 2
ZROW = T * N * 2


def _sc_body(xflat, sgb, bdlf, dstg, rawsrc, ones_h, z2d_h, z1d_h,
             acc_out, cnt_out, w2_out,
             bsrc_v, bdl_v, rows, acc_l, ones_v, p2src, p2dst,
             vals0, vals1, cnt_sh, w_sh, inv_sh,
             sem_g0, sem_o, sem_p):
    s = lax.axis_index("s")
    c = lax.axis_index("c")
    b32 = c * 16 + s

    pltpu.sync_copy(ones_h, ones_v)
    pltpu.sync_copy(z1d_h, cnt_sh.at[pl.ds(s * RPT, RPT)])
    pltpu.sync_copy(z1d_h, w_sh.at[pl.ds(s * RPT, RPT)])
    plsc.subcore_barrier()

    def zero_acc():
        for r in range(NB // CH):
            pltpu.sync_copy(z2d_h, acc_l.at[pl.ds(r * CH, CH), :])

    def edge_pass(t, h):
        zero_acc()

        def fire(si, grp):
            for b in range(K):
                ch = si * K + b
                pltpu.async_copy(xflat.at[bsrc_v.at[ch]],
                                 rows.at[pl.ds((grp * K + b) * CG, CG), :],
                                 sem_g0)

        def drain(si, grp):
            for b in range(K):
                ch = si * K + b
                pltpu.make_async_copy(
                    xflat.at[bsrc_v.at[ch]],
                    rows.at[pl.ds((grp * K + b) * CG, CG), :],
                    sem_g0).wait()

        def accumulate(si, grp):
            for b in range(K):
                ch = si * K + b

                def acc_body(g, _):
                    d16 = bdl_v[pl.ds(ch * CG + g * 16, 16)]
                    for k in range(16):
                        dl = d16[k]
                        e = g * 16 + k
                        for m in range(HD // 16):
                            v16 = rows[(grp * K + b) * CG + e,
                                       pl.ds(m * 16, 16)]
                            plsc.addupdate(acc_l.at[dl, pl.ds(m * 16, 16)],
                                           v16)
                    return 0

                lax.fori_loop(0, CG // 16, acc_body, 0)

        def refill_body(q8, _):
            qoff = q8 * OCT
            pltpu.sync_copy(sgb.at[h, t, b32, pl.ds(qoff, OCT)], bsrc_v)
            pltpu.sync_copy(bdlf.at[t, b32, pl.ds(qoff * CG, OCT * CG)],
                            bdl_v)
            fire(0, 0)

            def grp_body(si, _):
                grp = si % 2
                drain(si, grp)

                @pl.when(si < OCT // K - 1)
                def _():
                    fire(si + 1, 1 - grp)
                accumulate(si, grp)
                return 0

            lax.fori_loop(0, OCT // K, grp_body, 0)
            return 0

        lax.fori_loop(0, NCB // OCT, refill_body, 0)
        pltpu.sync_copy(acc_l, acc_out.at[h, t, pl.ds(b32 * NB, NB), :])

    def t_body(t, _):
        def ones_body(i, _):
            pltpu.async_copy(ones_v, cnt_sh.at[p2dst.at[i]], sem_o, add=True)
            return 0

        for hb in range(2):
            pltpu.sync_copy(
                dstg.at[t, pl.ds(s * OPC + hb * P2C, P2C)], p2dst)
            lax.fori_loop(0, P2C, ones_body, 0)
            pltpu.make_async_copy(
                dstg.at[t, pl.ds(s * OPC + hb * P2C, P2C)], p2dst,
                sem_o).wait()

        edge_pass(t, 0)
        plsc.subcore_barrier()

        @pl.when(c == 0)
        def _():
            pltpu.sync_copy(cnt_sh.at[pl.ds(s * RPT, RPT)],
                            cnt_out.at[pl.ds(t * NP + s * RPT, RPT)])

        for k in range(RPT // CH):
            off = s * RPT + k * CH
            pltpu.sync_copy(cnt_sh.at[pl.ds(off, CH)], vals0)
            for m in range(CH // 16):
                c16 = vals0[pl.ds(m * 16, 16)]
                gidx = off + m * 16 + lax.iota(jnp.int32, 16)
                vals1[pl.ds(m * 16, 16)] = jnp.where(
                    gidx < N, 1.0 / jnp.maximum(c16, 1.0), 0.0)
            pltpu.sync_copy(vals1, inv_sh.at[pl.ds(off, CH)])
        plsc.subcore_barrier()
        pltpu.sync_copy(z1d_h, cnt_sh.at[pl.ds(s * RPT, RPT)])

        KW = P2C // 2
        pltpu.sync_copy(dstg.at[t, pl.ds(b32 * P2C, P2C)], p2dst)
        pltpu.sync_copy(rawsrc.at[t, pl.ds(b32 * P2C, P2C)], p2src)
        for hb in range(2):
            for v in range(KW):
                pltpu.async_copy(inv_sh.at[p2dst.at[hb * KW + v]],
                                 rows.at[v, :], sem_g0)
            for v in range(KW):
                pltpu.make_async_copy(inv_sh.at[p2dst.at[hb * KW + v]],
                                      rows.at[v, :], sem_g0).wait()
            for v in range(KW):
                pltpu.async_copy(rows.at[v, :],
                                 w_sh.at[p2src.at[hb * KW + v]],
                                 sem_p, add=True)
            for v in range(KW):
                pltpu.make_async_copy(rows.at[v, :],
                                      w_sh.at[p2src.at[hb * KW + v]],
                                      sem_p).wait()
        plsc.subcore_barrier()

        pltpu.sync_copy(w_sh.at[pl.ds(s * RPT, RPT)],
                        w2_out.at[pl.ds((t * 2 + c) * NP + s * RPT, RPT)])
        pltpu.sync_copy(z1d_h, w_sh.at[pl.ds(s * RPT, RPT)])

        edge_pass(t, 1)
        plsc.subcore_barrier()
        return 0

    lax.fori_loop(0, T, t_body, 0)


def _sc_aggregate(xflat, sgb, bdl, dstg, rawsrc):
    mesh = plsc.VectorSubcoreMesh(core_axis_name="c", subcore_axis_name="s",
                                  num_cores=2, num_subcores=16)
    fn = functools.partial(
        pl.kernel,
        out_type=[
            jax.ShapeDtypeStruct((2, T, NP, HD), jnp.float32),
            jax.ShapeDtypeStruct((T * NP,), jnp.float32),
            jax.ShapeDtypeStruct((T * 2 * NP,), jnp.float32),
        ],
        mesh=mesh,
        scratch_types=[
            pltpu.VMEM((OCT, CG), jnp.int32),
            pltpu.VMEM((OCT * CG,), jnp.int32),
            pltpu.VMEM((2 * K * CG, HD), jnp.float32),
            pltpu.VMEM((NB, HD), jnp.float32),
            pltpu.VMEM((CH,), jnp.float32),
            pltpu.VMEM((P2C, CH), jnp.int32),
            pltpu.VMEM((P2C, CH), jnp.int32),
            pltpu.VMEM((CH,), jnp.float32),
            pltpu.VMEM((CH,), jnp.float32),
            pltpu.VMEM_SHARED((NP,), jnp.float32),
            pltpu.VMEM_SHARED((NP,), jnp.float32),
            pltpu.VMEM_SHARED((NP,), jnp.float32),
            pltpu.SemaphoreType.DMA,
            pltpu.SemaphoreType.DMA,
            pltpu.SemaphoreType.DMA,
        ],
        compiler_params=pltpu.CompilerParams(needs_layout_passes=False),
    )(_sc_body)
    ones_h = jnp.ones((CH,), jnp.float32)
    z2d_h = jnp.zeros((CH, HD), jnp.float32)
    z1d_h = jnp.zeros((RPT,), jnp.float32)
    return fn(xflat, sgb, bdl, dstg, rawsrc, ones_h, z2d_h, z1d_h)


BN = 1024


def _tc_a_body(x_ref, acc_ref, cnt_ref, w2_ref, wl1t_ref, wr1t_ref, b1_ref,
               out_ref):
    nb = pl.program_id(1)
    rowid = nb * BN + lax.broadcasted_iota(jnp.int32, (BN, 1), 0)
    valid = rowid < N
    x = jnp.where(valid, x_ref[0], 0.0)
    cnt = cnt_ref[0, 0]
    inv = 1.0 / jnp.maximum(cnt, 1.0)
    acc = jnp.concatenate([acc_ref[0, 0], acc_ref[1, 0]], axis=-1)
    mean = jnp.where(valid, acc * inv[:, None], 0.0)
    w = jnp.where(valid[:, 0], w2_ref[0, 0, :] + w2_ref[0, 1, :], 0.0)
    pre = (jnp.dot(mean, wl1t_ref[...], preferred_element_type=jnp.float32)
           + jnp.dot(x, wr1t_ref[...], preferred_element_type=jnp.float32)
           + b1_ref[...])
    p = jnp.maximum(pre, 0.0)
    pm = jnp.where(valid, p, 0.0)
    s0 = jnp.sum(pm, axis=0, keepdims=True)
    s1 = jnp.sum(pm * w[:, None], axis=0, keepdims=True)
    contrib = jnp.concatenate([s0, s1], axis=0) * (1.0 / N)

    @pl.when(nb == 0)
    def _():
        out_ref[0] = contrib

    @pl.when(nb > 0)
    def _():
        out_ref[0] = out_ref[0] + contrib


def _tc_a(x_seq, acc, cnt3, w23, wl1t, wr1t, b1r):
    nblk = NP // BN
    return pl.pallas_call(
        _tc_a_body,
        grid=(T, nblk),
        in_specs=[
            pl.BlockSpec((1, BN, D), lambda t, nb: (t, nb, 0)),
            pl.BlockSpec((2, 1, BN, HD), lambda t, nb: (0, t, nb, 0)),
            pl.BlockSpec((1, 1, BN), lambda t, nb: (t, 0, nb)),
            pl.BlockSpec((1, 2, BN), lambda t, nb: (t, 0, nb)),
            pl.BlockSpec((D, H), lambda t, nb: (0, 0)),
            pl.BlockSpec((D, H), lambda t, nb: (0, 0)),
            pl.BlockSpec((1, H), lambda t, nb: (0, 0)),
        ],
        out_specs=pl.BlockSpec((1, 2, H), lambda t, nb: (t, 0, 0)),
        out_shape=jax.ShapeDtypeStruct((T, 2, H), jnp.float32),
    )(x_seq, acc, cnt3, w23, wl1t, wr1t, b1r)


def _tc_b_body(ab_ref, wl2t_ref, wr2t_ref, b2_ref, wiht_ref, whht_ref,
               bih_ref, bhh_ref, woutt_ref, bout_ref, out_ref):
    ab = ab_ref[...]
    a_all = ab[:, 0, :]
    b_all = ab[:, 1, :]
    seq = (jnp.dot(b_all, wl2t_ref[...], preferred_element_type=jnp.float32)
           + jnp.dot(a_all, wr2t_ref[...], preferred_element_type=jnp.float32)
           + b2_ref[...])
    h = jnp.zeros((1, H), jnp.float32)
    c = jnp.zeros((1, H), jnp.float32)
    for t in range(T):
        g = (jnp.dot(seq[t:t + 1, :], wiht_ref[...],
                     preferred_element_type=jnp.float32) + bih_ref[...]
             + jnp.dot(h, whht_ref[...],
                       preferred_element_type=jnp.float32) + bhh_ref[...])
        gi = jax.nn.sigmoid(g[:, 0:H])
        gf = jax.nn.sigmoid(g[:, H:2 * H])
        gg = jnp.tanh(g[:, 2 * H:3 * H])
        go = jax.nn.sigmoid(g[:, 3 * H:4 * H])
        c = gf * c + gi * gg
        h = go * jnp.tanh(c)
    out_ref[...] = (jnp.dot(h, woutt_ref[...],
                            preferred_element_type=jnp.float32)
                    + bout_ref[...])


def _tc_b(ab, wl2t, wr2t, b2r, wiht, whht, bihr, bhhr, woutt, boutr):
    return pl.pallas_call(
        _tc_b_body,
        out_shape=jax.ShapeDtypeStruct((1, O), jnp.float32),
    )(ab, wl2t, wr2t, b2r, wiht, whht, bihr, bhhr, woutt, boutr)


def kernel(x_seq, edge_index_seq, Wl1, Wr1, b1, Wl2, Wr2, b2,
           W_ih, W_hh, b_ih, b_hh, W_out, b_out):
    src = edge_index_seq[:, 0, :]
    dst = edge_index_seq[:, 1, :]
    toff = (jnp.arange(T, dtype=jnp.int32) * N)[:, None]

    order = jnp.argsort(dst, axis=-1)
    ssorted = jnp.take_along_axis(src, order, axis=-1)
    dsorted = jnp.take_along_axis(dst, order, axis=-1)
    bin_lo = (jnp.arange(NBINS, dtype=jnp.int32) * NB)[None, :, None]
    starts = jnp.sum(dsorted[:, None, :] < bin_lo, axis=-1)
    ends = jnp.concatenate(
        [starts[:, 1:], jnp.full((T, 1), E, jnp.int32)], axis=1)
    slot = jnp.arange(CAP, dtype=jnp.int32)[None, None, :]
    idxm = starts[:, :, None] + slot
    validb = idxm < ends[:, :, None]
    idxc = jnp.clip(idxm, 0, E - 1).reshape(T, NBINS * CAP)
    bs = jnp.take_along_axis(ssorted, idxc, axis=-1).reshape(T, NBINS, CAP)
    bd = jnp.take_along_axis(dsorted, idxc, axis=-1).reshape(T, NBINS, CAP)
    sgl = (toff[:, None, :] + bs) * 2
    sgb = jnp.stack([jnp.where(validb, sgl, ZROW),
                     jnp.where(validb, sgl + 1, ZROW)],
                    axis=0).reshape(2, T, NBINS, NCB, CG)
    bdl = jnp.where(
        validb,
        bd - (jnp.arange(NBINS, dtype=jnp.int32) * NB)[None, :, None],
        0).reshape(T, NBINS, CAP)

    pad = EP - E
    padpos = jnp.arange(pad, dtype=jnp.int32)
    srcp = jnp.concatenate([src, jnp.zeros((T, pad), jnp.int32)], axis=1)
    dstp = jnp.concatenate(
        [dst, jnp.broadcast_to(N + padpos % (NP - N), (T, pad))], axis=1)
    dstg = dstp.reshape(T, EP // CH, CH)
    rawsrc = srcp.reshape(T, EP // CH, CH)
    xflat = jnp.concatenate(
        [x_seq.reshape(T * N * 2, HD),
         jnp.zeros((8, HD), jnp.float32)], axis=0)

    acc, cnt, w2 = _sc_aggregate(xflat, sgb, bdl, dstg, rawsrc)

    ab = _tc_a(x_seq, acc, cnt.reshape(T, 1, NP), w2.reshape(T, 2, NP),
               Wl1.T, Wr1.T, b1.reshape(1, H))
    out = _tc_b(ab, Wl2.T, Wr2.T, b2.reshape(1, H),
                W_ih.T, W_hh.T, b_ih.reshape(1, 4 * H),
                b_hh.reshape(1, 4 * H), W_out.T, b_out.reshape(1, O))
    return out


# speedup vs baseline: 1.1821x; 1.1821x over previous
import functools

import jax
import jax.numpy as jnp
from jax import lax
from jax.experimental import pallas as pl
from jax.experimental.pallas import tpu as pltpu
from jax.experimental.pallas import tpu_sc as plsc

T, N, E, D, H, O = 8, 10000, 160000, 256, 256, 128
NP = 10240
NBINS = 16
NB = 640
CAP = 10752
EP = 163840
CH = 128
CG = 32
K = 3
NCB = CAP // CG
OCT = 24
P2C = EP // 16 // CH
RPT = NP // 16
HD = D // 2
ZROW = T * N * 2


def _sc_body(xflat, sgb, bdlf, dstg, rawsrc, ones_h, z2d_h, z1d_h,
             acc_out, cnt_out, w2_out,
             bsrc_v, bdl_v, rows, acc_l, ones_v, p2src, p2dst,
             vals0, vals1, cnt_sh, w_sh, inv_sh,
             sem_g0, sem_o, sem_p):
    s = lax.axis_index("s")

    pltpu.sync_copy(ones_h, ones_v)
    pltpu.sync_copy(z1d_h, cnt_sh.at[pl.ds(s * RPT, RPT)])
    pltpu.sync_copy(z1d_h, w_sh.at[pl.ds(s * RPT, RPT)])
    plsc.subcore_barrier()

    def zero_acc():
        for r in range(NB // CH):
            pltpu.sync_copy(z2d_h, acc_l.at[pl.ds(r * CH, CH), :])

    def edge_pass(t, h):
        zero_acc()

        def fire(si, grp):
            for b in range(K):
                ch = si * K + b
                pltpu.async_copy(xflat.at[bsrc_v.at[ch]],
                                 rows.at[pl.ds((grp * K + b) * CG, CG), :],
                                 sem_g0)

        def drain(si, grp):
            for b in range(K):
                ch = si * K + b
                pltpu.make_async_copy(
                    xflat.at[bsrc_v.at[ch]],
                    rows.at[pl.ds((grp * K + b) * CG, CG), :],
                    sem_g0).wait()

        def accumulate(si, grp):
            for b in range(K):
                ch = si * K + b

                def acc_body(g, _):
                    d16 = bdl_v[pl.ds(ch * CG + g * 16, 16)]
                    for k in range(16):
                        dl = d16[k]
                        e = g * 16 + k
                        for m in range(HD // 16):
                            v16 = rows[(grp * K + b) * CG + e,
                                       pl.ds(m * 16, 16)]
                            plsc.addupdate(acc_l.at[dl, pl.ds(m * 16, 16)],
                                           v16)
                    return 0

                lax.fori_loop(0, CG // 16, acc_body, 0)

        def refill_body(q8, _):
            qoff = q8 * OCT
            pltpu.sync_copy(sgb.at[h, t, s, pl.ds(qoff, OCT)], bsrc_v)
            pltpu.sync_copy(bdlf.at[t, s, pl.ds(qoff * CG, OCT * CG)], bdl_v)
            fire(0, 0)

            def grp_body(si, _):
                grp = si % 2
                drain(si, grp)

                @pl.when(si < OCT // K - 1)
                def _():
                    fire(si + 1, 1 - grp)
                accumulate(si, grp)
                return 0

            lax.fori_loop(0, OCT // K, grp_body, 0)
            return 0

        lax.fori_loop(0, NCB // OCT, refill_body, 0)
        pltpu.sync_copy(acc_l, acc_out.at[h, t, pl.ds(s * NB, NB), :])

    def t_body(t, _):
        pltpu.sync_copy(dstg.at[t, pl.ds(s * P2C, P2C)], p2dst)

        def ones_body(i, _):
            pltpu.async_copy(ones_v, cnt_sh.at[p2dst.at[i]], sem_o, add=True)
            return 0

        lax.fori_loop(0, P2C, ones_body, 0)

        edge_pass(t, 0)

        pltpu.make_async_copy(
            dstg.at[t, pl.ds(s * P2C, P2C)], p2dst, sem_o).wait()
        plsc.subcore_barrier()

        pltpu.sync_copy(cnt_sh.at[pl.ds(s * RPT, RPT)],
                        cnt_out.at[pl.ds(t * NP + s * RPT, RPT)])

        for k in range(RPT // CH):
            off = s * RPT + k * CH
            pltpu.sync_copy(cnt_sh.at[pl.ds(off, CH)], vals0)
            for m in range(CH // 16):
                c16 = vals0[pl.ds(m * 16, 16)]
                gidx = off + m * 16 + lax.iota(jnp.int32, 16)
                vals1[pl.ds(m * 16, 16)] = jnp.where(
                    gidx < N, 1.0 / jnp.maximum(c16, 1.0), 0.0)
            pltpu.sync_copy(vals1, inv_sh.at[pl.ds(off, CH)])
        plsc.subcore_barrier()
        pltpu.sync_copy(z1d_h, cnt_sh.at[pl.ds(s * RPT, RPT)])

        KW = 8
        for hb in range(2):
            pltpu.sync_copy(
                rawsrc.at[t, pl.ds(s * P2C + hb * (P2C // 2), P2C // 2)],
                p2src)

            def p2_body(jj, _):
                for v in range(KW):
                    ch = jj * KW + v
                    pltpu.async_copy(
                        inv_sh.at[p2dst.at[hb * (P2C // 2) + ch]],
                        rows.at[v, :], sem_g0)
                for v in range(KW):
                    ch = jj * KW + v
                    pltpu.make_async_copy(
                        inv_sh.at[p2dst.at[hb * (P2C // 2) + ch]],
                        rows.at[v, :], sem_g0).wait()
                for v in range(KW):
                    ch = jj * KW + v
                    pltpu.async_copy(rows.at[v, :], w_sh.at[p2src.at[ch]],
                                     sem_p, add=True)
                for v in range(KW):
                    ch = jj * KW + v
                    pltpu.make_async_copy(rows.at[v, :],
                                          w_sh.at[p2src.at[ch]], sem_p).wait()
                return 0

            lax.fori_loop(0, P2C // 2 // KW, p2_body, 0)
        plsc.subcore_barrier()

        pltpu.sync_copy(w_sh.at[pl.ds(s * RPT, RPT)],
                        w2_out.at[pl.ds(t * NP + s * RPT, RPT)])
        pltpu.sync_copy(z1d_h, w_sh.at[pl.ds(s * RPT, RPT)])

        edge_pass(t, 1)
        plsc.subcore_barrier()
        return 0

    lax.fori_loop(0, T, t_body, 0)


def _sc_aggregate(xflat, sgb, bdl, dstg, rawsrc):
    mesh = plsc.VectorSubcoreMesh(core_axis_name="c", subcore_axis_name="s",
                                  num_cores=1, num_subcores=16)
    fn = functools.partial(
        pl.kernel,
        out_type=[
            jax.ShapeDtypeStruct((2, T, NP, HD), jnp.float32),
            jax.ShapeDtypeStruct((T * NP,), jnp.float32),
            jax.ShapeDtypeStruct((T * NP,), jnp.float32),
        ],
        mesh=mesh,
        scratch_types=[
            pltpu.VMEM((OCT, CG), jnp.int32),
            pltpu.VMEM((OCT * CG,), jnp.int32),
            pltpu.VMEM((2 * K * CG, HD), jnp.float32),
            pltpu.VMEM((NB, HD), jnp.float32),
            pltpu.VMEM((CH,), jnp.float32),
            pltpu.VMEM((P2C // 2, CH), jnp.int32),
            pltpu.VMEM((P2C, CH), jnp.int32),
            pltpu.VMEM((CH,), jnp.float32),
            pltpu.VMEM((CH,), jnp.float32),
            pltpu.VMEM_SHARED((NP,), jnp.float32),
            pltpu.VMEM_SHARED((NP,), jnp.float32),
            pltpu.VMEM_SHARED((NP,), jnp.float32),
            pltpu.SemaphoreType.DMA,
            pltpu.SemaphoreType.DMA,
            pltpu.SemaphoreType.DMA,
        ],
        compiler_params=pltpu.CompilerParams(needs_layout_passes=False),
    )(_sc_body)
    ones_h = jnp.ones((CH,), jnp.float32)
    z2d_h = jnp.zeros((CH, HD), jnp.float32)
    z1d_h = jnp.zeros((RPT,), jnp.float32)
    return fn(xflat, sgb, bdl, dstg, rawsrc, ones_h, z2d_h, z1d_h)


BN = 1024


def _tc_a_body(x_ref, acc_ref, cnt_ref, w2_ref, wl1t_ref, wr1t_ref, b1_ref,
               out_ref):
    nb = pl.program_id(1)
    rowid = nb * BN + lax.broadcasted_iota(jnp.int32, (BN, 1), 0)
    valid = rowid < N
    x = jnp.where(valid, x_ref[0], 0.0)
    cnt = cnt_ref[0, 0]
    inv = 1.0 / jnp.maximum(cnt, 1.0)
    acc = jnp.concatenate([acc_ref[0, 0], acc_ref[1, 0]], axis=-1)
    mean = jnp.where(valid, acc * inv[:, None], 0.0)
    w = jnp.where(valid[:, 0], w2_ref[0, 0, :], 0.0)
    pre = (jnp.dot(mean, wl1t_ref[...], preferred_element_type=jnp.float32)
           + jnp.dot(x, wr1t_ref[...], preferred_element_type=jnp.float32)
           + b1_ref[...])
    p = jnp.maximum(pre, 0.0)
    pm = jnp.where(valid, p, 0.0)
    s0 = jnp.sum(pm, axis=0, keepdims=True)
    s1 = jnp.sum(pm * w[:, None], axis=0, keepdims=True)
    contrib = jnp.concatenate([s0, s1], axis=0) * (1.0 / N)

    @pl.when(nb == 0)
    def _():
        out_ref[0] = contrib

    @pl.when(nb > 0)
    def _():
        out_ref[0] = out_ref[0] + contrib


def _tc_a(x_seq, acc, cnt3, w23, wl1t, wr1t, b1r):
    nblk = NP // BN
    return pl.pallas_call(
        _tc_a_body,
        grid=(T, nblk),
        in_specs=[
            pl.BlockSpec((1, BN, D), lambda t, nb: (t, nb, 0)),
            pl.BlockSpec((2, 1, BN, HD), lambda t, nb: (0, t, nb, 0)),
            pl.BlockSpec((1, 1, BN), lambda t, nb: (t, 0, nb)),
            pl.BlockSpec((1, 1, BN), lambda t, nb: (t, 0, nb)),
            pl.BlockSpec((D, H), lambda t, nb: (0, 0)),
            pl.BlockSpec((D, H), lambda t, nb: (0, 0)),
            pl.BlockSpec((1, H), lambda t, nb: (0, 0)),
        ],
        out_specs=pl.BlockSpec((1, 2, H), lambda t, nb: (t, 0, 0)),
        out_shape=jax.ShapeDtypeStruct((T, 2, H), jnp.float32),
    )(x_seq, acc, cnt3, w23, wl1t, wr1t, b1r)


def _tc_b_body(ab_ref, wl2t_ref, wr2t_ref, b2_ref, wiht_ref, whht_ref,
               bih_ref, bhh_ref, woutt_ref, bout_ref, out_ref):
    ab = ab_ref[...]
    a_all = ab[:, 0, :]
    b_all = ab[:, 1, :]
    seq = (jnp.dot(b_all, wl2t_ref[...], preferred_element_type=jnp.float32)
           + jnp.dot(a_all, wr2t_ref[...], preferred_element_type=jnp.float32)
           + b2_ref[...])
    h = jnp.zeros((1, H), jnp.float32)
    c = jnp.zeros((1, H), jnp.float32)
    for t in range(T):
        g = (jnp.dot(seq[t:t + 1, :], wiht_ref[...],
                     preferred_element_type=jnp.float32) + bih_ref[...]
             + jnp.dot(h, whht_ref[...],
                       preferred_element_type=jnp.float32) + bhh_ref[...])
        gi = jax.nn.sigmoid(g[:, 0:H])
        gf = jax.nn.sigmoid(g[:, H:2 * H])
        gg = jnp.tanh(g[:, 2 * H:3 * H])
        go = jax.nn.sigmoid(g[:, 3 * H:4 * H])
        c = gf * c + gi * gg
        h = go * jnp.tanh(c)
    out_ref[...] = (jnp.dot(h, woutt_ref[...],
                            preferred_element_type=jnp.float32)
                    + bout_ref[...])


def _tc_b(ab, wl2t, wr2t, b2r, wiht, whht, bihr, bhhr, woutt, boutr):
    return pl.pallas_call(
        _tc_b_body,
        out_shape=jax.ShapeDtypeStruct((1, O), jnp.float32),
    )(ab, wl2t, wr2t, b2r, wiht, whht, bihr, bhhr, woutt, boutr)


def kernel(x_seq, edge_index_seq, Wl1, Wr1, b1, Wl2, Wr2, b2,
           W_ih, W_hh, b_ih, b_hh, W_out, b_out):
    src = edge_index_seq[:, 0, :]
    dst = edge_index_seq[:, 1, :]
    toff = (jnp.arange(T, dtype=jnp.int32) * N)[:, None]

    order = jnp.argsort(dst, axis=-1)
    ssorted = jnp.take_along_axis(src, order, axis=-1)
    dsorted = jnp.take_along_axis(dst, order, axis=-1)
    bin_lo = (jnp.arange(NBINS, dtype=jnp.int32) * NB)[None, :, None]
    starts = jnp.sum(dsorted[:, None, :] < bin_lo, axis=-1)
    ends = jnp.concatenate(
        [starts[:, 1:], jnp.full((T, 1), E, jnp.int32)], axis=1)
    slot = jnp.arange(CAP, dtype=jnp.int32)[None, None, :]
    idxm = starts[:, :, None] + slot
    validb = idxm < ends[:, :, None]
    idxc = jnp.clip(idxm, 0, E - 1).reshape(T, NBINS * CAP)
    bs = jnp.take_along_axis(ssorted, idxc, axis=-1).reshape(T, NBINS, CAP)
    bd = jnp.take_along_axis(dsorted, idxc, axis=-1).reshape(T, NBINS, CAP)
    sgl = (toff[:, None, :] + bs) * 2
    sgb = jnp.stack([jnp.where(validb, sgl, ZROW),
                     jnp.where(validb, sgl + 1, ZROW)],
                    axis=0).reshape(2, T, NBINS, NCB, CG)
    bdl = jnp.where(
        validb,
        bd - (jnp.arange(NBINS, dtype=jnp.int32) * NB)[None, :, None],
        0).reshape(T, NBINS, CAP)

    pad = EP - E
    padpos = jnp.arange(pad, dtype=jnp.int32)
    srcp = jnp.concatenate([src, jnp.zeros((T, pad), jnp.int32)], axis=1)
    dstp = jnp.concatenate(
        [dst, jnp.broadcast_to(N + padpos % (NP - N), (T, pad))], axis=1)
    dstg = dstp.reshape(T, EP // CH, CH)
    rawsrc = srcp.reshape(T, EP // CH, CH)
    xflat = jnp.concatenate(
        [x_seq.reshape(T * N * 2, HD),
         jnp.zeros((8, HD), jnp.float32)], axis=0)

    acc, cnt, w2 = _sc_aggregate(xflat, sgb, bdl, dstg, rawsrc)

    ab = _tc_a(x_seq, acc, cnt.reshape(T, 1, NP), w2.reshape(T, 1, NP),
               Wl1.T, Wr1.T, b1.reshape(1, H))
    out = _tc_b(ab, Wl2.T, Wr2.T, b2.reshape(1, H),
                W_ih.T, W_hh.T, b_ih.reshape(1, 4 * H),
                b_hh.reshape(1, 4 * H), W_out.T, b_out.reshape(1, O))
    return out
